# baseline (speedup 1.0000x reference)
import jax
import jax.numpy as jnp
from jax.experimental import pallas as pl
from jax.experimental.pallas import tpu as pltpu

BN = 2048
NJ = 2


def _gmlvq_body(x_ref, w_ref, rc_ref, out_ref, wr_ref, w2_ref, x2_ref):
    rc = rc_ref[0, :]
    j = pl.program_id(1)

    @pl.when((pl.program_id(0) == 0) & (j == 0))
    def _prep():
        w32 = w_ref[...]
        wr_ref[...] = (w32 * (-2.0 * rc)[None, :]).astype(jnp.bfloat16)
        w2_ref[0, :] = jnp.sum(w32 * w32 * rc[None, :], axis=1)

    x32 = x_ref[...]

    @pl.when(j == 0)
    def _row_norms():
        x2_ref[:, 0] = jnp.sum(x32 * x32 * rc[None, :], axis=1)

    c = w_ref.shape[0]
    bc = c // NJ
    xb = x32.astype(jnp.bfloat16)
    cross = jax.lax.dot_general(
        xb, wr_ref[pl.ds(j * bc, bc), :], (((1,), (1,)), ((), ())),
        preferred_element_type=jnp.float32)
    w2j = w2_ref[0, pl.ds(j * bc, bc)]
    out_ref[...] = cross + x2_ref[:, 0][:, None] + w2j[None, :]


def kernel(X, W, r):
    n, f = X.shape
    c, _ = W.shape
    rc = jnp.clip(r, 1e-6, None).reshape(1, f)
    return pl.pallas_call(
        _gmlvq_body,
        grid=(n // BN, NJ),
        in_specs=[
            pl.BlockSpec((BN, f), lambda i, j: (i, 0)),
            pl.BlockSpec((c, f), lambda i, j: (0, 0)),
            pl.BlockSpec((1, f), lambda i, j: (0, 0)),
        ],
        out_specs=pl.BlockSpec((BN, c // NJ), lambda i, j: (i, j)),
        out_shape=jax.ShapeDtypeStruct((n, c), jnp.float32),
        scratch_shapes=[
            pltpu.VMEM((c, f), jnp.bfloat16),
            pltpu.VMEM((1, c), jnp.float32),
            pltpu.VMEM((BN, 1), jnp.float32),
        ],
        compiler_params=pltpu.CompilerParams(
            dimension_semantics=("arbitrary", "arbitrary"),
            vmem_limit_bytes=60 * 1024 * 1024),
    )(X, W, rc)


# speedup vs baseline: 1.5637x; 1.5637x over previous
import jax
import jax.numpy as jnp
from jax.experimental import pallas as pl
from jax.experimental.pallas import tpu as pltpu

BN = 2048


def _gmlvq_body(x_ref, w_ref, rc_ref, out_ref, wr_ref, w2_ref):
    rc = rc_ref[0, :]

    @pl.when(pl.program_id(0) == 0)
    def _prep():
        w32 = w_ref[...]
        wr_ref[...] = (w32 * (-2.0 * rc)[None, :]).astype(jnp.bfloat16)
        w2_ref[0, :] = jnp.sum(w32 * w32 * rc[None, :], axis=1)

    x32 = x_ref[...]
    xb = x32.astype(jnp.bfloat16)
    cross = jax.lax.dot_general(
        xb, wr_ref[...], (((1,), (1,)), ((), ())),
        preferred_element_type=jnp.float32)
    x2 = jnp.sum(x32 * x32 * rc[None, :], axis=1)
    out_ref[...] = cross + x2[:, None] + w2_ref[0, :][None, :]


def kernel(X, W, r):
    n, f = X.shape
    c, _ = W.shape
    rc = jnp.clip(r, 1e-6, None).reshape(1, f)
    return pl.pallas_call(
        _gmlvq_body,
        grid=(n // BN,),
        in_specs=[
            pl.BlockSpec((BN, f), lambda i: (i, 0)),
            pl.BlockSpec((c, f), lambda i: (0, 0)),
            pl.BlockSpec((1, f), lambda i: (0, 0)),
        ],
        out_specs=pl.BlockSpec((BN, c), lambda i: (i, 0)),
        out_shape=jax.ShapeDtypeStruct((n, c), jnp.float32),
        scratch_shapes=[
            pltpu.VMEM((c, f), jnp.bfloat16),
            pltpu.VMEM((1, c), jnp.float32),
        ],
        compiler_params=pltpu.CompilerParams(
            dimension_semantics=("arbitrary",),
            vmem_limit_bytes=60 * 1024 * 1024),
    )(X, W, rc)


# speedup vs baseline: 1.5657x; 1.0013x over previous
import jax
import jax.numpy as jnp
from jax.experimental import pallas as pl
from jax.experimental.pallas import tpu as pltpu

BN = 2048


def _gmlvq_body(x_ref, w_ref, rc_ref, out_ref, wr_ref, w2_ref):
    rc = jnp.clip(rc_ref[0, :], 1e-6, None)

    @pl.when(pl.program_id(0) == 0)
    def _prep():
        w32 = w_ref[...]
        wr_ref[...] = (w32 * (-2.0 * rc)[None, :]).astype(jnp.bfloat16)
        w2_ref[0, :] = jnp.sum(w32 * w32 * rc[None, :], axis=1)

    x32 = x_ref[...]
    xb = x32.astype(jnp.bfloat16)
    cross = jax.lax.dot_general(
        xb, wr_ref[...], (((1,), (1,)), ((), ())),
        preferred_element_type=jnp.float32)
    x2 = jnp.sum(x32 * x32 * rc[None, :], axis=1)
    out_ref[...] = cross + x2[:, None] + w2_ref[0, :][None, :]


def kernel(X, W, r):
    n, f = X.shape
    c, _ = W.shape
    rc = r.reshape(1, f)
    return pl.pallas_call(
        _gmlvq_body,
        grid=(n // BN,),
        in_specs=[
            pl.BlockSpec((BN, f), lambda i: (i, 0)),
            pl.BlockSpec((c, f), lambda i: (0, 0)),
            pl.BlockSpec((1, f), lambda i: (0, 0)),
        ],
        out_specs=pl.BlockSpec((BN, c), lambda i: (i, 0)),
        out_shape=jax.ShapeDtypeStruct((n, c), jnp.float32),
        scratch_shapes=[
            pltpu.VMEM((c, f), jnp.bfloat16),
            pltpu.VMEM((1, c), jnp.float32),
        ],
        compiler_params=pltpu.CompilerParams(
            dimension_semantics=("arbitrary",),
            vmem_limit_bytes=60 * 1024 * 1024),
    )(X, W, rc)
